# Initial kernel scaffold; baseline (speedup 1.0000x reference)
import jax
import jax.numpy as jnp
from jax.experimental import pallas as pl


def kernel(x, edge_index, W1, b1, W2, b2, W3, b3, g1, beta1, g2, beta2, g3, beta3, Wf, bf):
    raise NotImplementedError("write your pallas kernel here")


# speedup vs baseline: 11.6084x; 11.6084x over previous
import functools

import jax
import jax.numpy as jnp
from jax import lax
from jax.experimental import pallas as pl
from jax.experimental.pallas import tpu as pltpu
from jax.experimental.pallas import tpu_sc as plsc

N = 10000
E = 320000
D = 128
NC = 2
NS = 16
NW = NC * NS

EPW = E // NW
CHUNK = 80
NCHUNK = EPW // CHUNK

NP = 10240
NPT = NP // NS
ZR = 128
NZ = NPT // ZR


@functools.cache
def _make_deg_kernel():
    mesh = plsc.VectorSubcoreMesh(core_axis_name="c", subcore_axis_name="s")
    return functools.partial(
        pl.kernel, mesh=mesh,
        out_type=jax.ShapeDtypeStruct((NC, NP), jnp.float32),
        scratch_types=[
            pltpu.VMEM((EPW,), jnp.int32),
            pltpu.VMEM((NP,), jnp.float32),
            pltpu.VMEM((NPT,), jnp.float32),
            pltpu.VMEM((NPT,), jnp.float32),
            pltpu.VMEM_SHARED((NS * NP,), jnp.float32),
        ],
        compiler_params=pltpu.CompilerParams(needs_layout_passes=False),
    )(_deg_body)


def _deg_body(dst_hbm, out_hbm, didx, ldeg, accv, tmpv, deg_sh):
    c = lax.axis_index("c")
    s = lax.axis_index("s")
    wid = c * NS + s
    zf = jnp.zeros((16,), jnp.float32)

    def zero_ldeg(i, _):
        ldeg[pl.ds(i * 16, 16)] = zf
        return 0
    lax.fori_loop(0, NP // 16, zero_ldeg, 0)

    pltpu.sync_copy(dst_hbm.at[pl.ds(wid * EPW, EPW)], didx)
    ones = jnp.ones((16,), jnp.float32)

    def count(i, _):
        idx = didx[pl.ds(i * 16, 16)]
        plsc.addupdate_scatter(ldeg, [idx], ones)
        return 0
    lax.fori_loop(0, EPW // 16, count, 0)

    pltpu.sync_copy(ldeg, deg_sh.at[pl.ds(s * NP, NP)])
    plsc.subcore_barrier()

    def zero_acc(i, _):
        accv[pl.ds(i * 16, 16)] = zf
        return 0
    lax.fori_loop(0, NPT // 16, zero_acc, 0)

    for k in range(NS):
        pltpu.sync_copy(deg_sh.at[pl.ds(k * NP + s * NPT, NPT)], tmpv)

        def addk(i, _):
            accv[pl.ds(i * 16, 16)] = accv[pl.ds(i * 16, 16)] + tmpv[pl.ds(i * 16, 16)]
            return 0
        lax.fori_loop(0, NPT // 16, addk, 0)

    pltpu.sync_copy(accv, out_hbm.at[c].at[pl.ds(s * NPT, NPT)])


@functools.cache
def _make_agg_kernel():
    mesh = plsc.VectorSubcoreMesh(core_axis_name="c", subcore_axis_name="s")
    return functools.partial(
        pl.kernel, mesh=mesh,
        out_type=jax.ShapeDtypeStruct((NC, NP, D), jnp.float32),
        scratch_types=[
            pltpu.VMEM((CHUNK,), jnp.int32),
            pltpu.VMEM((CHUNK,), jnp.int32),
            pltpu.VMEM((CHUNK, D), jnp.float32),
            pltpu.VMEM((ZR, D), jnp.float32),
            pltpu.VMEM_SHARED((NP, D), jnp.float32),
            pltpu.SemaphoreType.DMA,
        ],
        compiler_params=pltpu.CompilerParams(needs_layout_passes=False),
    )(_agg_body)


def _agg_body(h_hbm, src_hbm, dst_hbm, out_hbm, sidx, didx, rows, zbuf, agg_sh, sem):
    c = lax.axis_index("c")
    s = lax.axis_index("s")
    wid = c * NS + s
    zf = jnp.zeros((16,), jnp.float32)

    def zero_zbuf(i, _):
        zbuf[i // 8, pl.ds((i % 8) * 16, 16)] = zf
        return 0
    lax.fori_loop(0, ZR * 8, zero_zbuf, 0)

    def zero_sh(j, _):
        pltpu.sync_copy(zbuf, agg_sh.at[pl.ds(s * NPT + j * ZR, ZR)])
        return 0
    lax.fori_loop(0, NZ, zero_sh, 0)
    plsc.subcore_barrier()

    base = wid * EPW

    def chunk_body(i, _):
        off = base + i * CHUNK
        pltpu.sync_copy(src_hbm.at[pl.ds(off, CHUNK)], sidx)
        pltpu.sync_copy(dst_hbm.at[pl.ds(off, CHUNK)], didx)
        pltpu.async_copy(h_hbm.at[sidx], rows, sem).wait()
        pltpu.sync_copy(rows, agg_sh.at[didx], add=True)
        return 0
    lax.fori_loop(0, NCHUNK, chunk_body, 0)
    plsc.subcore_barrier()

    def writeout(j, _):
        r0 = s * NPT + j * ZR
        pltpu.sync_copy(agg_sh.at[pl.ds(r0, ZR)], out_hbm.at[c].at[pl.ds(r0, ZR)])
        return 0
    lax.fori_loop(0, NZ, writeout, 0)


_DN = (((1,), (1,)), ((), ()))


def _tc1_body(x_ref, w_ref, degp_ref, hp_ref, dcol_ref):
    deg = degp_ref[0] + degp_ref[1] + 1.0
    dcol = lax.rsqrt(deg)[:N]
    m = lax.dot_general(x_ref[...], w_ref[...], _DN,
                        preferred_element_type=jnp.float32)
    hp_ref[...] = m * dcol
    dcol_ref[...] = dcol


_tc1 = pl.pallas_call(
    _tc1_body,
    out_shape=[jax.ShapeDtypeStruct((N, D), jnp.float32),
               jax.ShapeDtypeStruct((N, 1), jnp.float32)],
)


def _bn_relu(u, g, beta):
    mean = jnp.mean(u, axis=0)
    var = jnp.mean((u - mean) ** 2, axis=0)
    return jnp.maximum((u - mean) * lax.rsqrt(var + 1e-5) * g + beta, 0.0)


def _tc_mid_body(aggp_ref, hp_ref, dcol_ref, b_ref, g_ref, beta_ref, w_ref,
                 h_ref, hpn_ref):
    dcol = dcol_ref[...]
    u = (aggp_ref[0, :N] + aggp_ref[1, :N] + hp_ref[...]) * dcol + b_ref[...]
    h = _bn_relu(u, g_ref[...], beta_ref[...])
    h_ref[...] = h
    m = lax.dot_general(h, w_ref[...], _DN, preferred_element_type=jnp.float32)
    hpn_ref[...] = m * dcol


_tc_mid = pl.pallas_call(
    _tc_mid_body,
    out_shape=[jax.ShapeDtypeStruct((N, D), jnp.float32),
               jax.ShapeDtypeStruct((N, D), jnp.float32)],
)


def _tc_out_body(aggp_ref, hp_ref, dcol_ref, b_ref, g_ref, beta_ref,
                 h1_ref, h2_ref, wf_ref, bf_ref, out_ref):
    dcol = dcol_ref[...]
    u = (aggp_ref[0, :N] + aggp_ref[1, :N] + hp_ref[...]) * dcol + b_ref[...]
    h3 = _bn_relu(u, g_ref[...], beta_ref[...])
    wf = wf_ref[...]
    acc = lax.dot_general(h1_ref[...], wf[:, 0:D], _DN,
                          preferred_element_type=jnp.float32)
    acc = acc + lax.dot_general(h2_ref[...], wf[:, D:2 * D], _DN,
                                preferred_element_type=jnp.float32)
    acc = acc + lax.dot_general(h3, wf[:, 2 * D:3 * D], _DN,
                                preferred_element_type=jnp.float32)
    out_ref[...] = acc + bf_ref[...]


_tc_out = pl.pallas_call(
    _tc_out_body,
    out_shape=jax.ShapeDtypeStruct((N, 64), jnp.float32),
)


def kernel(x, edge_index, W1, b1, W2, b2, W3, b3, g1, beta1, g2, beta2,
           g3, beta3, Wf, bf):
    src = edge_index[0]
    dst = edge_index[1]
    deg_k = _make_deg_kernel()
    agg_k = _make_agg_kernel()
    degp = deg_k(dst).reshape(NC, NP, 1)
    h1p, dcol = _tc1(x, W1, degp)
    agg1 = agg_k(h1p, src, dst)
    h1, h2p = _tc_mid(agg1, h1p, dcol, b1, g1, beta1, W2)
    agg2 = agg_k(h2p, src, dst)
    h2, h3p = _tc_mid(agg2, h2p, dcol, b2, g2, beta2, W3)
    agg3 = agg_k(h3p, src, dst)
    out = _tc_out(agg3, h3p, dcol, b3, g3, beta3, h1, h2, Wf, bf)
    return out
